# baseline (speedup 1.0000x reference)
import jax
import jax.numpy as jnp
from jax import lax
from jax.experimental import pallas as pl
from jax.experimental.pallas import tpu as pltpu
from jax.experimental.pallas import tpu_sc as plsc

_B = 16384
_N = 200
_D = 64
_V = 1000000
_NC = 2
_NS = 16
_NW = _NC * _NS
_TN = _N // 8
_TB = _B // 128
_TJW = _TB // _NW


def _gather_body(idx_hbm, tab_hbm, out_hbm,
                 idxb, gidx0, gidx1, rows0, rows1, wout0, wout1,
                 sg0, sg1, so0, so1):
    wid = lax.axis_index("s") * _NC + lax.axis_index("c")
    tj0 = wid * _TJW

    gidx = (gidx0, gidx1)
    rows = (rows0, rows1)
    wout = (wout0, wout1)
    sg = (sg0, sg1)
    so = (so0, so1)

    viota = lax.iota(jnp.int32, 16)

    def build_gidx(s, b):
        tb = jnp.bitwise_and(b, 3)
        ni = jnp.right_shift(b, 2)
        for v in range(8):
            gidx[s][pl.ds(16 * v, 16)] = (
                idxb[tb, ni, pl.ds(16 * v, 16)] * 2)

    def gather_cp(s):
        return pltpu.make_async_copy(tab_hbm.at[gidx[s]], rows[s], sg[s])

    def out_cp(s, n, tj):
        return pltpu.make_async_copy(wout[s], out_hbm.at[n, :, tj], so[s])

    def transpose(s):
        rv = rows[s]
        wv = wout[s]

        @plsc.parallel_loop(0, 8, unroll=2)
        def d8_body(d8):
            col0 = jnp.full((16,), d8 * 8, jnp.int32)
            for dd in range(8):
                col = col0 + dd
                vals = [plsc.load_gather(rv, [viota + 16 * j, col])
                        for j in range(8)]
                for j in range(8):
                    wv[d8, dd, pl.ds(16 * j, 16)] = vals[j]

    def step(tn, b, s, pf):
        n = tn * 8 + jnp.right_shift(b, 2)
        tj = tj0 + jnp.bitwise_and(b, 3)
        gather_cp(s).wait()
        out_cp(s, n, tj).wait()
        transpose(s)
        out_cp(s, n, tj).start()
        if pf is not None:
            build_gidx(s, pf)
            gather_cp(s).start()

    out_cp(0, 0, tj0).start()
    out_cp(1, 0, tj0).start()

    def tn_body(tn, carry):
        pltpu.sync_copy(idx_hbm.at[tn, pl.ds(tj0, _TJW)], idxb)
        build_gidx(0, jnp.int32(0))
        gather_cp(0).start()
        build_gidx(1, jnp.int32(1))
        gather_cp(1).start()

        def blk(m, c):
            b0 = 2 * m
            step(tn, b0, 0, b0 + 2)
            step(tn, b0 + 1, 1, b0 + 3)
            return c

        lax.fori_loop(0, 15, blk, 0)
        step(tn, jnp.int32(30), 0, None)
        step(tn, jnp.int32(31), 1, None)
        return carry

    lax.fori_loop(0, _TN, tn_body, 0)

    pltpu.make_async_copy(wout[0], out_hbm.at[_N - 8, :, tj0], so[0]).wait()
    pltpu.make_async_copy(wout[1], out_hbm.at[_N - 8, :, tj0], so[1]).wait()


def kernel(location_ids, table):
    idx4 = location_ids.T.reshape(_TN, 8, 128, 128).transpose(0, 2, 1, 3)
    tabp = jnp.pad(table, ((0, 0), (0, 64))).reshape(2 * _V, _D)
    mesh = plsc.VectorSubcoreMesh(core_axis_name="c", subcore_axis_name="s")
    out5 = pl.kernel(
        _gather_body,
        out_type=jax.ShapeDtypeStruct((_N, 8, 128, 8, 128), jnp.float32),
        mesh=mesh,
        scratch_types=[
            pltpu.VMEM((_TJW, 8, 128), jnp.int32),
            pltpu.VMEM((128,), jnp.int32),
            pltpu.VMEM((128,), jnp.int32),
            pltpu.VMEM((128, _D), jnp.float32),
            pltpu.VMEM((128, _D), jnp.float32),
            pltpu.VMEM((8, 8, 128), jnp.float32),
            pltpu.VMEM((8, 8, 128), jnp.float32),
            pltpu.SemaphoreType.DMA,
            pltpu.SemaphoreType.DMA,
            pltpu.SemaphoreType.DMA,
            pltpu.SemaphoreType.DMA,
        ],
        compiler_params=pltpu.CompilerParams(
            use_tc_tiling_on_sc=False, needs_layout_passes=False),
    )(idx4, tabp)
    return out5.transpose((2, 4, 0, 1, 3)).reshape(_B, _N, _D)


# speedup vs baseline: 2.8486x; 2.8486x over previous
import jax
import jax.numpy as jnp
from jax import lax
from jax.experimental import pallas as pl
from jax.experimental.pallas import tpu as pltpu
from jax.experimental.pallas import tpu_sc as plsc

_B = 16384
_N = 200
_D = 64
_V = 1000000
_NC = 2
_NS = 16
_NW = _NC * _NS
_TN = _N // 8
_TB = _B // 128
_TJW = _TB // _NW


def _gather_body(idx_hbm, tab_hbm, out_hbm,
                 idxb, gidx0, gidx1, rows0, rows1, wout0, wout1,
                 sg0, sg1, so0, so1):
    wid = lax.axis_index("s") * _NC + lax.axis_index("c")
    tj0 = wid * _TJW

    gidx = (gidx0, gidx1)
    rows = (rows0, rows1)
    wout = (wout0, wout1)
    sg = (sg0, sg1)
    so = (so0, so1)

    viota = lax.iota(jnp.int32, 16)

    def build_gidx(s, b):
        tb = jnp.bitwise_and(b, 3)
        ni = jnp.right_shift(b, 2)
        for v in range(8):
            gidx[s][pl.ds(16 * v, 16)] = (
                idxb[tb, ni, pl.ds(16 * v, 16)] * 2)

    def gather_cp(s):
        return pltpu.make_async_copy(tab_hbm.at[gidx[s]], rows[s], sg[s])

    def out_cp(s, n, tj):
        return pltpu.make_async_copy(wout[s], out_hbm.at[n, :, tj], so[s])

    def transpose(s):
        rv = rows[s]
        wv = wout[s]

        @plsc.parallel_loop(0, 16)
        def k_body(k):
            patk = jnp.bitwise_and(viota + k, 15)
            vti_p = jnp.right_shift(patk, 3)
            vdi = jnp.bitwise_and(patk, 7)
            for j in range(8):
                vrow = viota + 16 * j
                for c in range(4):
                    vcol = patk + 16 * c
                    vti = vti_p + 2 * c
                    val = plsc.load_gather(rv, [vrow, vcol])
                    plsc.store_scatter(wv, [vti, vdi, vrow], val)

    def step(tn, b, s, pf):
        n = tn * 8 + jnp.right_shift(b, 2)
        tj = tj0 + jnp.bitwise_and(b, 3)
        gather_cp(s).wait()
        out_cp(s, n, tj).wait()
        transpose(s)
        out_cp(s, n, tj).start()
        if pf is not None:
            build_gidx(s, pf)
            gather_cp(s).start()

    out_cp(0, 0, tj0).start()
    out_cp(1, 0, tj0).start()

    def tn_body(tn, carry):
        pltpu.sync_copy(idx_hbm.at[tn, pl.ds(tj0, _TJW)], idxb)
        build_gidx(0, jnp.int32(0))
        gather_cp(0).start()
        build_gidx(1, jnp.int32(1))
        gather_cp(1).start()

        def blk(m, c):
            b0 = 2 * m
            step(tn, b0, 0, b0 + 2)
            step(tn, b0 + 1, 1, b0 + 3)
            return c

        lax.fori_loop(0, 15, blk, 0)
        step(tn, jnp.int32(30), 0, None)
        step(tn, jnp.int32(31), 1, None)
        return carry

    lax.fori_loop(0, _TN, tn_body, 0)

    pltpu.make_async_copy(wout[0], out_hbm.at[_N - 8, :, tj0], so[0]).wait()
    pltpu.make_async_copy(wout[1], out_hbm.at[_N - 8, :, tj0], so[1]).wait()


def kernel(location_ids, table):
    idx4 = location_ids.T.reshape(_TN, 8, 128, 128).transpose(0, 2, 1, 3)
    tabp = jnp.pad(table, ((0, 0), (0, 64))).reshape(2 * _V, _D)
    mesh = plsc.VectorSubcoreMesh(core_axis_name="c", subcore_axis_name="s")
    out5 = pl.kernel(
        _gather_body,
        out_type=jax.ShapeDtypeStruct((_N, 8, 128, 8, 128), jnp.float32),
        mesh=mesh,
        scratch_types=[
            pltpu.VMEM((_TJW, 8, 128), jnp.int32),
            pltpu.VMEM((128,), jnp.int32),
            pltpu.VMEM((128,), jnp.int32),
            pltpu.VMEM((128, _D), jnp.float32),
            pltpu.VMEM((128, _D), jnp.float32),
            pltpu.VMEM((8, 8, 128), jnp.float32),
            pltpu.VMEM((8, 8, 128), jnp.float32),
            pltpu.SemaphoreType.DMA,
            pltpu.SemaphoreType.DMA,
            pltpu.SemaphoreType.DMA,
            pltpu.SemaphoreType.DMA,
        ],
        compiler_params=pltpu.CompilerParams(
            use_tc_tiling_on_sc=False, needs_layout_passes=False),
    )(idx4, tabp)
    return out5.transpose((2, 4, 0, 1, 3)).reshape(_B, _N, _D)


# speedup vs baseline: 3.3384x; 1.1719x over previous
import jax
import jax.numpy as jnp
from jax import lax
from jax.experimental import pallas as pl
from jax.experimental.pallas import tpu as pltpu
from jax.experimental.pallas import tpu_sc as plsc

_B = 16384
_N = 200
_D = 64
_V = 1000000
_NC = 2
_NS = 16
_NW = _NC * _NS
_TN = _N // 8
_TB = _B // 128
_TJW = _TB // _NW


def _gather_body(idx_hbm, tab_hbm, out_hbm,
                 idxb, gidx0, gidx1, rows0, rows1, wout0, wout1,
                 sg0, sg1, so0, so1):
    wid = lax.axis_index("s") * _NC + lax.axis_index("c")
    tj0 = wid * _TJW

    gidx = (gidx0, gidx1)
    rows = (rows0, rows1)
    wout = (wout0, wout1)
    sg = (sg0, sg1)
    so = (so0, so1)

    viota = lax.iota(jnp.int32, 16)

    def build_gidx(s, b):
        tb = jnp.bitwise_and(b, 3)
        ni = jnp.right_shift(b, 2)
        for v in range(8):
            gidx[s][pl.ds(16 * v, 16)] = (
                idxb[tb, ni, pl.ds(16 * v, 16)] * 2)

    def gather_cp(s):
        return pltpu.make_async_copy(tab_hbm.at[gidx[s]], rows[s], sg[s])

    def out_cp(s, n, tj):
        return pltpu.make_async_copy(wout[s], out_hbm.at[n, :, tj], so[s])

    def transpose(s):
        rv = rows[s]
        wv = wout[s]

        @plsc.parallel_loop(0, 16, unroll=2)
        def k_body(k):
            patk = jnp.bitwise_and(viota + k, 15)
            vti_p = jnp.right_shift(patk, 3)
            vdi = jnp.bitwise_and(patk, 7)
            for j in range(8):
                vrow = viota + 16 * j
                for c in range(4):
                    vcol = patk + 16 * c
                    vti = vti_p + 2 * c
                    val = plsc.load_gather(rv, [vrow, vcol])
                    plsc.store_scatter(wv, [vti, vdi, vrow], val)

    def step(tn, b, s, pf):
        n = tn * 8 + jnp.right_shift(b, 2)
        tj = tj0 + jnp.bitwise_and(b, 3)
        gather_cp(s).wait()
        out_cp(s, n, tj).wait()
        transpose(s)
        out_cp(s, n, tj).start()
        if pf is not None:
            build_gidx(s, pf)
            gather_cp(s).start()

    out_cp(0, 0, tj0).start()
    out_cp(1, 0, tj0).start()

    def tn_body(tn, carry):
        pltpu.sync_copy(idx_hbm.at[tn, pl.ds(tj0, _TJW)], idxb)
        build_gidx(0, jnp.int32(0))
        gather_cp(0).start()
        build_gidx(1, jnp.int32(1))
        gather_cp(1).start()

        def blk(m, c):
            b0 = 2 * m
            step(tn, b0, 0, b0 + 2)
            step(tn, b0 + 1, 1, b0 + 3)
            return c

        lax.fori_loop(0, 15, blk, 0)
        step(tn, jnp.int32(30), 0, None)
        step(tn, jnp.int32(31), 1, None)
        return carry

    lax.fori_loop(0, _TN, tn_body, 0)

    pltpu.make_async_copy(wout[0], out_hbm.at[_N - 8, :, tj0], so[0]).wait()
    pltpu.make_async_copy(wout[1], out_hbm.at[_N - 8, :, tj0], so[1]).wait()


def kernel(location_ids, table):
    idx4 = location_ids.T.reshape(_TN, 8, 128, 128).transpose(0, 2, 1, 3)
    tabp = jnp.pad(table, ((0, 0), (0, 64))).reshape(2 * _V, _D)
    mesh = plsc.VectorSubcoreMesh(core_axis_name="c", subcore_axis_name="s")
    out5 = pl.kernel(
        _gather_body,
        out_type=jax.ShapeDtypeStruct((_N, 8, 128, 8, 128), jnp.float32),
        mesh=mesh,
        scratch_types=[
            pltpu.VMEM((_TJW, 8, 128), jnp.int32),
            pltpu.VMEM((128,), jnp.int32),
            pltpu.VMEM((128,), jnp.int32),
            pltpu.VMEM((128, _D), jnp.float32),
            pltpu.VMEM((128, _D), jnp.float32),
            pltpu.VMEM((8, 8, 128), jnp.float32),
            pltpu.VMEM((8, 8, 128), jnp.float32),
            pltpu.SemaphoreType.DMA,
            pltpu.SemaphoreType.DMA,
            pltpu.SemaphoreType.DMA,
            pltpu.SemaphoreType.DMA,
        ],
        compiler_params=pltpu.CompilerParams(
            use_tc_tiling_on_sc=False, needs_layout_passes=False),
    )(idx4, tabp)
    return out5.transpose((2, 4, 0, 1, 3)).reshape(_B, _N, _D)


# speedup vs baseline: 3.3932x; 1.0164x over previous
import jax
import jax.numpy as jnp
from jax import lax
from jax.experimental import pallas as pl
from jax.experimental.pallas import tpu as pltpu
from jax.experimental.pallas import tpu_sc as plsc

_B = 16384
_N = 200
_D = 64
_V = 1000000
_NC = 2
_NS = 16
_NW = _NC * _NS
_TN = _N // 8
_TB = _B // 128
_TJW = _TB // _NW


def _gather_body(idx_hbm, tab_hbm, out_hbm,
                 idxb, gidx0, gidx1, rows0, rows1, wout0, wout1,
                 sg0, sg1, so0, so1):
    wid = lax.axis_index("s") * _NC + lax.axis_index("c")
    tj0 = wid * _TJW

    gidx = (gidx0, gidx1)
    rows = (rows0, rows1)
    wout = (wout0, wout1)
    sg = (sg0, sg1)
    so = (so0, so1)

    viota = lax.iota(jnp.int32, 16)

    def build_gidx(s, b):
        tb = jnp.bitwise_and(b, 3)
        ni = jnp.right_shift(b, 2)
        for v in range(8):
            gidx[s][pl.ds(16 * v, 16)] = (
                idxb[tb, ni, pl.ds(16 * v, 16)] * 2)

    def gather_cp(s):
        return pltpu.make_async_copy(tab_hbm.at[gidx[s]], rows[s], sg[s])

    def out_cp(s, n, tj):
        return pltpu.make_async_copy(wout[s], out_hbm.at[n, :, tj], so[s])

    def transpose(s):
        rv = rows[s]
        wv = wout[s]

        @plsc.parallel_loop(0, 16, unroll=4)
        def k_body(k):
            patk = jnp.bitwise_and(viota + k, 15)
            vti_p = jnp.right_shift(patk, 3)
            vdi = jnp.bitwise_and(patk, 7)
            for j in range(8):
                vrow = viota + 16 * j
                for c in range(4):
                    vcol = patk + 16 * c
                    vti = vti_p + 2 * c
                    val = plsc.load_gather(rv, [vrow, vcol])
                    plsc.store_scatter(wv, [vti, vdi, vrow], val)

    def step(tn, b, s, pf):
        n = tn * 8 + jnp.right_shift(b, 2)
        tj = tj0 + jnp.bitwise_and(b, 3)
        gather_cp(s).wait()
        out_cp(s, n, tj).wait()
        transpose(s)
        out_cp(s, n, tj).start()
        if pf is not None:
            build_gidx(s, pf)
            gather_cp(s).start()

    out_cp(0, 0, tj0).start()
    out_cp(1, 0, tj0).start()

    def tn_body(tn, carry):
        pltpu.sync_copy(idx_hbm.at[tn, pl.ds(tj0, _TJW)], idxb)
        build_gidx(0, jnp.int32(0))
        gather_cp(0).start()
        build_gidx(1, jnp.int32(1))
        gather_cp(1).start()

        def blk(m, c):
            b0 = 2 * m
            step(tn, b0, 0, b0 + 2)
            step(tn, b0 + 1, 1, b0 + 3)
            return c

        lax.fori_loop(0, 15, blk, 0)
        step(tn, jnp.int32(30), 0, None)
        step(tn, jnp.int32(31), 1, None)
        return carry

    lax.fori_loop(0, _TN, tn_body, 0)

    pltpu.make_async_copy(wout[0], out_hbm.at[_N - 8, :, tj0], so[0]).wait()
    pltpu.make_async_copy(wout[1], out_hbm.at[_N - 8, :, tj0], so[1]).wait()


def kernel(location_ids, table):
    idx4 = location_ids.T.reshape(_TN, 8, 128, 128).transpose(0, 2, 1, 3)
    tabp = jnp.pad(table, ((0, 0), (0, 64))).reshape(2 * _V, _D)
    mesh = plsc.VectorSubcoreMesh(core_axis_name="c", subcore_axis_name="s")
    out5 = pl.kernel(
        _gather_body,
        out_type=jax.ShapeDtypeStruct((_N, 8, 128, 8, 128), jnp.float32),
        mesh=mesh,
        scratch_types=[
            pltpu.VMEM((_TJW, 8, 128), jnp.int32),
            pltpu.VMEM((128,), jnp.int32),
            pltpu.VMEM((128,), jnp.int32),
            pltpu.VMEM((128, _D), jnp.float32),
            pltpu.VMEM((128, _D), jnp.float32),
            pltpu.VMEM((8, 8, 128), jnp.float32),
            pltpu.VMEM((8, 8, 128), jnp.float32),
            pltpu.SemaphoreType.DMA,
            pltpu.SemaphoreType.DMA,
            pltpu.SemaphoreType.DMA,
            pltpu.SemaphoreType.DMA,
        ],
        compiler_params=pltpu.CompilerParams(
            use_tc_tiling_on_sc=False, needs_layout_passes=False),
    )(idx4, tabp)
    return out5.transpose((2, 4, 0, 1, 3)).reshape(_B, _N, _D)
